# baseline (speedup 1.0000x reference)
import functools

import jax
import jax.numpy as jnp
from jax import lax
from jax.experimental import pallas as pl
from jax.experimental.pallas import tpu as pltpu
from jax.experimental.pallas import tpu_sc as plsc

N_NODES = 10000
N_EDGES = 320000
D_FEAT = 128
DH = D_FEAT // 2

NC, NS = 2, 16
NW = NC * NS
CHUNK = 112
NBUF = 4
NCHUNK = 92
EPT = NCHUNK * CHUNK
PAD_EDGES = NW * EPT
ROWS_PAD = 10112
RPT = ROWS_PAD // NS
DUMMY_DST = N_NODES


def _scale_body(nf_ref, d_ref, o_ref):
    o_ref[0] = nf_ref[:, :DH] * d_ref[...]
    o_ref[1] = nf_ref[:, DH:] * d_ref[...]


def _combine_body(p_ref, dg_ref, o_ref):
    o_ref[:, :DH] = (p_ref[0, 0] + p_ref[0, 1]) * dg_ref[...]
    o_ref[:, DH:] = (p_ref[1, 0] + p_ref[1, 1]) * dg_ref[...]


_sc_mesh = plsc.VectorSubcoreMesh(core_axis_name="c", subcore_axis_name="s")


@functools.partial(
    pl.kernel,
    out_type=jax.ShapeDtypeStruct((2, NC, ROWS_PAD, DH), jnp.float32),
    mesh=_sc_mesh,
    scratch_types=[
        pltpu.VMEM((NCHUNK + NBUF, CHUNK), jnp.int32),
        pltpu.VMEM((NCHUNK, CHUNK), jnp.int32),
        [pltpu.VMEM((CHUNK, DH), jnp.float32)] * NBUF,
        pltpu.VMEM_SHARED((ROWS_PAD, DH), jnp.float32),
        [pltpu.SemaphoreType.DMA] * NBUF,
        [pltpu.SemaphoreType.DMA] * NBUF,
    ],
    compiler_params=pltpu.CompilerParams(use_tc_tiling_on_sc=False),
)
def _sc_aggregate(x_hbm, src_hbm, dst_hbm, zeros_hbm, out_hbm,
                  src_v, dst_v, rows, acc, gsems, ssems):
    cid = lax.axis_index("c")
    sid = lax.axis_index("s")

    pltpu.sync_copy(src_hbm.at[cid, sid], src_v)
    pltpu.sync_copy(dst_hbm.at[cid, sid], dst_v)

    for p in range(2):
        pltpu.sync_copy(zeros_hbm.at[pl.ds(sid * RPT, RPT)],
                        acc.at[pl.ds(sid * RPT, RPT)])
        plsc.subcore_barrier()

        xp = x_hbm.at[p]
        for b in range(NBUF):
            pltpu.async_copy(xp.at[src_v.at[b]], rows[b], gsems[b])

        def step(q, carry):
            g0 = NBUF * q
            for b in range(NBUF):
                pltpu.make_async_copy(xp.at[src_v.at[g0 + b]],
                                      rows[b], gsems[b]).wait()
                pltpu.sync_copy(rows[b], acc.at[dst_v.at[g0 + b]], add=True)
                pltpu.async_copy(xp.at[src_v.at[g0 + NBUF + b]],
                                 rows[b], gsems[b])
            return carry

        lax.fori_loop(0, NCHUNK // NBUF, step, 0)
        for b in range(NBUF):
            pltpu.make_async_copy(xp.at[src_v.at[NCHUNK + b]],
                                  rows[b], gsems[b]).wait()

        plsc.subcore_barrier()
        pltpu.sync_copy(acc.at[pl.ds(sid * RPT, RPT)],
                        out_hbm.at[p, cid, pl.ds(sid * RPT, RPT)])


def kernel(node_f, out_d, in_dg, edge_index):
    ei = edge_index.astype(jnp.int32)
    src = jnp.concatenate(
        [ei[0], jnp.zeros((PAD_EDGES - N_EDGES,), jnp.int32)])
    dst = jnp.concatenate(
        [ei[1], jnp.full((PAD_EDGES - N_EDGES,), DUMMY_DST, jnp.int32)])
    src = src.reshape(NC, NS, NCHUNK, CHUNK)
    src = jnp.concatenate(
        [src, jnp.zeros((NC, NS, NBUF, CHUNK), jnp.int32)], axis=2)
    dst = dst.reshape(NC, NS, NCHUNK, CHUNK)
    zeros = jnp.zeros((ROWS_PAD, DH), jnp.float32)

    x = pl.pallas_call(
        _scale_body,
        out_shape=jax.ShapeDtypeStruct((2, N_NODES, DH), jnp.float32),
    )(node_f, out_d)

    partials = _sc_aggregate(x, src, dst, zeros)

    rst = pl.pallas_call(
        _combine_body,
        out_shape=jax.ShapeDtypeStruct((N_NODES, D_FEAT), jnp.float32),
    )(partials[:, :, :N_NODES], in_dg)
    return rst


# speedup vs baseline: 2.0728x; 2.0728x over previous
import functools

import jax
import jax.numpy as jnp
from jax import lax
from jax.experimental import pallas as pl
from jax.experimental.pallas import tpu as pltpu
from jax.experimental.pallas import tpu_sc as plsc

N_NODES = 10000
N_EDGES = 320000
D_FEAT = 128
DH = D_FEAT // 2

NC, NS = 2, 16
NW = NC * NS
CHUNK = 112
NBUF = 2
NCHUNK = 90
EPT = NCHUNK * CHUNK
PAD_EDGES = NW * EPT
ROWS_PAD = 10112
RPT = ROWS_PAD // NS
DUMMY_DST = N_NODES


def _scale_body(nf_ref, d_ref, o_ref):
    o_ref[0] = nf_ref[:, :DH] * d_ref[...]
    o_ref[1] = nf_ref[:, DH:] * d_ref[...]


def _combine_body(p_ref, dg_ref, o_ref):
    o_ref[:, :DH] = (p_ref[0, 0] + p_ref[0, 1]) * dg_ref[...]
    o_ref[:, DH:] = (p_ref[1, 0] + p_ref[1, 1]) * dg_ref[...]


_sc_mesh = plsc.VectorSubcoreMesh(core_axis_name="c", subcore_axis_name="s")


@functools.partial(
    pl.kernel,
    out_type=jax.ShapeDtypeStruct((2, NC, ROWS_PAD, DH), jnp.float32),
    mesh=_sc_mesh,
    scratch_types=[
        pltpu.VMEM((NCHUNK + NBUF, CHUNK), jnp.int32),
        pltpu.VMEM((NCHUNK, CHUNK), jnp.int32),
        [pltpu.VMEM((CHUNK, DH // 2), jnp.int32)] * NBUF,
        pltpu.VMEM((CHUNK, DH), jnp.float32),
        pltpu.VMEM_SHARED((ROWS_PAD, DH), jnp.float32),
        [pltpu.SemaphoreType.DMA] * NBUF,
        [pltpu.SemaphoreType.DMA] * NBUF,
    ],
    compiler_params=pltpu.CompilerParams(use_tc_tiling_on_sc=False,
                                         needs_layout_passes=False),
)
def _sc_aggregate(x_hbm, src_hbm, dst_hbm, zeros_hbm, out_hbm,
                  src_v, dst_v, rows, conv, acc, gsems, ssems):
    cid = lax.axis_index("c")
    sid = lax.axis_index("s")

    pltpu.sync_copy(src_hbm.at[cid, sid], src_v)
    pltpu.sync_copy(dst_hbm.at[cid, sid], dst_v)

    for p in range(2):
        pltpu.sync_copy(zeros_hbm.at[pl.ds(sid * RPT, RPT)],
                        acc.at[pl.ds(sid * RPT, RPT)])
        plsc.subcore_barrier()

        xp = x_hbm.at[p]
        for b in range(NBUF):
            pltpu.async_copy(xp.at[src_v.at[b]], rows[b], gsems[b])

        def step(q, carry):
            g0 = NBUF * q
            for b in range(NBUF):
                pltpu.make_async_copy(xp.at[src_v.at[g0 + b]],
                                      rows[b], gsems[b]).wait()

                def conv_row(r, c2):
                    for k in range(DH // 32):
                        w = rows[b][r, pl.ds(16 * k, 16)]
                        conv[r, pl.ds(32 * k, 16)] = plsc.bitcast(
                            jnp.left_shift(w, 16), jnp.float32)
                        conv[r, pl.ds(32 * k + 16, 16)] = plsc.bitcast(
                            jnp.bitwise_and(w, jnp.int32(-65536)), jnp.float32)
                    return c2

                lax.fori_loop(0, CHUNK, conv_row, 0)
                pltpu.sync_copy(conv, acc.at[dst_v.at[g0 + b]], add=True)
                pltpu.async_copy(xp.at[src_v.at[g0 + NBUF + b]],
                                 rows[b], gsems[b])
            return carry

        lax.fori_loop(0, NCHUNK // NBUF, step, 0)
        for b in range(NBUF):
            pltpu.make_async_copy(xp.at[src_v.at[NCHUNK + b]],
                                  rows[b], gsems[b]).wait()

        plsc.subcore_barrier()
        pltpu.sync_copy(acc.at[pl.ds(sid * RPT, RPT)],
                        out_hbm.at[p, cid, pl.ds(sid * RPT, RPT)])


def kernel(node_f, out_d, in_dg, edge_index):
    ei = edge_index.astype(jnp.int32)
    src = jnp.concatenate(
        [ei[0], jnp.zeros((PAD_EDGES - N_EDGES,), jnp.int32)])
    dst = jnp.concatenate(
        [ei[1], jnp.full((PAD_EDGES - N_EDGES,), DUMMY_DST, jnp.int32)])
    src = src.reshape(NC, NS, NCHUNK, CHUNK)
    src = jnp.concatenate(
        [src, jnp.zeros((NC, NS, NBUF, CHUNK), jnp.int32)], axis=2)
    dst = dst.reshape(NC, NS, NCHUNK, CHUNK)
    zeros = jnp.zeros((ROWS_PAD, DH), jnp.float32)

    x = pl.pallas_call(
        _scale_body,
        out_shape=jax.ShapeDtypeStruct((2, N_NODES, DH), jnp.float32),
    )(node_f, out_d)

    x16 = (x.reshape(2, N_NODES, DH // 32, 2, 16)
             .transpose(0, 1, 2, 4, 3)
             .reshape(2, N_NODES, DH // 2, 2)
             .astype(jnp.bfloat16))
    xpacked = jax.lax.bitcast_convert_type(x16, jnp.int32)

    partials = _sc_aggregate(xpacked, src, dst, zeros)

    rst = pl.pallas_call(
        _combine_body,
        out_shape=jax.ShapeDtypeStruct((N_NODES, D_FEAT), jnp.float32),
    )(partials[:, :, :N_NODES], in_dg)
    return rst


# speedup vs baseline: 2.7658x; 1.3343x over previous
import functools

import jax
import jax.numpy as jnp
from jax import lax
from jax.experimental import pallas as pl
from jax.experimental.pallas import tpu as pltpu
from jax.experimental.pallas import tpu_sc as plsc

N_NODES = 10000
N_EDGES = 320000
D_FEAT = 128
DH = D_FEAT // 2

NC, NS = 2, 16
NW = NC * NS
CHUNK = 112
NBUF = 2
NCHUNK = 90
EPT = NCHUNK * CHUNK
PAD_EDGES = NW * EPT
ROWS_PAD = 10112
RPT = ROWS_PAD // NS
DUMMY_DST = N_NODES


def _scale_body(nf_ref, d_ref, o_ref):
    o_ref[0] = nf_ref[:, :DH] * d_ref[...]
    o_ref[1] = nf_ref[:, DH:] * d_ref[...]


def _combine_body(p_ref, dg_ref, o_ref):
    o_ref[:, :DH] = (p_ref[0, 0] + p_ref[0, 1]) * dg_ref[...]
    o_ref[:, DH:] = (p_ref[1, 0] + p_ref[1, 1]) * dg_ref[...]


_sc_mesh = plsc.VectorSubcoreMesh(core_axis_name="c", subcore_axis_name="s")


@functools.partial(
    pl.kernel,
    out_type=jax.ShapeDtypeStruct((2, NC, ROWS_PAD, DH), jnp.float32),
    mesh=_sc_mesh,
    scratch_types=[
        pltpu.VMEM((NCHUNK + NBUF, CHUNK), jnp.int32),
        pltpu.VMEM((NCHUNK, CHUNK), jnp.int32),
        [pltpu.VMEM((CHUNK, DH // 2), jnp.int32)] * NBUF,
        [pltpu.VMEM((CHUNK, DH), jnp.float32)] * NBUF,
        pltpu.VMEM_SHARED((ROWS_PAD, DH), jnp.float32),
        [pltpu.SemaphoreType.DMA] * NBUF,
        [pltpu.SemaphoreType.DMA] * NBUF,
    ],
    compiler_params=pltpu.CompilerParams(use_tc_tiling_on_sc=False,
                                         needs_layout_passes=False),
)
def _sc_aggregate(x_hbm, src_hbm, dst_hbm, zeros_hbm, out_hbm,
                  src_v, dst_v, rows, conv, acc, gsems, ssems):
    cid = lax.axis_index("c")
    sid = lax.axis_index("s")

    pltpu.sync_copy(src_hbm.at[cid, sid], src_v)
    pltpu.sync_copy(dst_hbm.at[cid, sid], dst_v)

    for p in range(2):
        pltpu.sync_copy(zeros_hbm.at[pl.ds(sid * RPT, RPT)],
                        acc.at[pl.ds(sid * RPT, RPT)])
        plsc.subcore_barrier()

        xp = x_hbm.at[p]

        def widen(rb, cb):
            @functools.partial(plsc.parallel_loop, 0, CHUNK, unroll=8)
            def _conv_row(r):
                for k in range(DH // 32):
                    w = rb[r, pl.ds(16 * k, 16)]
                    cb[r, pl.ds(32 * k, 16)] = plsc.bitcast(
                        jnp.left_shift(w, 16), jnp.float32)
                    cb[r, pl.ds(32 * k + 16, 16)] = plsc.bitcast(
                        jnp.bitwise_and(w, jnp.int32(-65536)), jnp.float32)

        for b in range(NBUF):
            pltpu.async_copy(xp.at[src_v.at[b]], rows[b], gsems[b])
        for b in range(NBUF):
            pltpu.make_async_copy(xp.at[src_v.at[b]], rows[b], gsems[b]).wait()
            widen(rows[b], conv[b])
            pltpu.async_copy(conv[b], acc.at[dst_v.at[b]], ssems[b], add=True)
            pltpu.async_copy(xp.at[src_v.at[NBUF + b]], rows[b], gsems[b])

        def step(q, carry):
            g0 = NBUF * q
            for b in range(NBUF):
                pltpu.make_async_copy(xp.at[src_v.at[g0 + b]],
                                      rows[b], gsems[b]).wait()
                pltpu.make_async_copy(conv[b], acc.at[dst_v.at[g0 + b]],
                                      ssems[b]).wait()
                widen(rows[b], conv[b])
                pltpu.async_copy(conv[b], acc.at[dst_v.at[g0 + b]],
                                 ssems[b], add=True)
                pltpu.async_copy(xp.at[src_v.at[g0 + NBUF + b]],
                                 rows[b], gsems[b])
            return carry

        lax.fori_loop(1, NCHUNK // NBUF, step, 0)
        for b in range(NBUF):
            pltpu.make_async_copy(xp.at[src_v.at[NCHUNK + b]],
                                  rows[b], gsems[b]).wait()
            pltpu.make_async_copy(conv[b], acc.at[dst_v.at[0]],
                                  ssems[b]).wait()

        plsc.subcore_barrier()
        pltpu.sync_copy(acc.at[pl.ds(sid * RPT, RPT)],
                        out_hbm.at[p, cid, pl.ds(sid * RPT, RPT)])


def kernel(node_f, out_d, in_dg, edge_index):
    ei = edge_index.astype(jnp.int32)
    src = jnp.concatenate(
        [ei[0], jnp.zeros((PAD_EDGES - N_EDGES,), jnp.int32)])
    dst = jnp.concatenate(
        [ei[1], jnp.full((PAD_EDGES - N_EDGES,), DUMMY_DST, jnp.int32)])
    src = src.reshape(NC, NS, NCHUNK, CHUNK)
    src = jnp.concatenate(
        [src, jnp.zeros((NC, NS, NBUF, CHUNK), jnp.int32)], axis=2)
    dst = dst.reshape(NC, NS, NCHUNK, CHUNK)
    zeros = jnp.zeros((ROWS_PAD, DH), jnp.float32)

    x = pl.pallas_call(
        _scale_body,
        out_shape=jax.ShapeDtypeStruct((2, N_NODES, DH), jnp.float32),
    )(node_f, out_d)

    x16 = (x.reshape(2, N_NODES, DH // 32, 2, 16)
             .transpose(0, 1, 2, 4, 3)
             .reshape(2, N_NODES, DH // 2, 2)
             .astype(jnp.bfloat16))
    xpacked = jax.lax.bitcast_convert_type(x16, jnp.int32)

    partials = _sc_aggregate(xpacked, src, dst, zeros)

    rst = pl.pallas_call(
        _combine_body,
        out_shape=jax.ShapeDtypeStruct((N_NODES, D_FEAT), jnp.float32),
    )(partials[:, :, :N_NODES], in_dg)
    return rst
